# baseline (speedup 1.0000x reference)
import functools

import jax
import jax.numpy as jnp
from jax import lax
from jax.experimental import pallas as pl
from jax.experimental.pallas import tpu as pltpu

B = 32
S = 1024
D = 128
H = 64
G = 4 * H
K = 256
NB = 8
T = S // NB

_ARB = pltpu.CompilerParams(dimension_semantics=("arbitrary",))


W2 = 8 * H


def _proj_body(two_stream, *refs):
    if two_stream:
        (xfa, xfb, xra, xrb, wf, wr, pc_ref) = refs
        xf = jnp.concatenate([xfa[...], xfb[...]], axis=-1)
        xr = jnp.concatenate([xra[...], xrb[...]], axis=-1)
    else:
        (xfa, xra, wf, wr, pc_ref) = refs
        xf = xfa[...]
        xr = xra[...]
    din = xf.shape[-1]
    pf = jnp.dot(xf.reshape(T * B, din), wf[...]).reshape(T, B, W2)
    pr = jnp.dot(xr.reshape(T * B, din), wr[...]).reshape(T, B, W2)
    pc_ref[...] = pf + pr


def _rec_body(emit_flipped, pc_ref, bihb, bhhb, wbd, *refs):
    if emit_flipped:
        of_ref, or_ref, off_ref, orf_ref, h_s, c_s = refs
    else:
        of_ref, or_ref, h_s, c_s = refs
    i = pl.program_id(0)

    @pl.when(i == 0)
    def _init():
        h_s[...] = jnp.zeros_like(h_s)
        c_s[...] = jnp.zeros_like(c_s)

    def body(t, carry):
        h, c = carry
        g = pc_ref[t] + jnp.dot(h, wbd[...])
        g = g + bihb[...]
        g = g + bhhb[...]
        ii = g[:, 0:2 * H]
        ff = g[:, 2 * H:4 * H]
        gg = g[:, 4 * H:6 * H]
        oo = g[:, 6 * H:8 * H]
        c2 = jax.nn.sigmoid(ff) * c + jax.nn.sigmoid(ii) * jnp.tanh(gg)
        h2 = jax.nn.sigmoid(oo) * jnp.tanh(c2)
        of_ref[t] = h2[:, 0:H]
        or_ref[T - 1 - t] = h2[:, H:2 * H]
        if emit_flipped:
            off_ref[T - 1 - t] = h2[:, 0:H]
            orf_ref[t] = h2[:, H:2 * H]
        return h2, c2

    h, c = lax.fori_loop(0, T, body, (h_s[...], c_s[...]))
    h_s[...] = h
    c_s[...] = c


def _bilstm_layer(fwd_arrs, rev_arrs, din, args, emit_flipped):
    n_in = len(fwd_arrs)
    w = din // n_in
    in_specs = ([pl.BlockSpec((T, B, w), lambda i: (i, 0, 0))
                 for _ in range(2 * n_in)])
    operands = list(fwd_arrs) + list(rev_arrs)
    wf, wr, bihb, bhhb, wbd = args
    in_specs += [
        pl.BlockSpec((din, W2), lambda i: (0, 0)),
        pl.BlockSpec((din, W2), lambda i: (0, 0)),
    ]
    operands += [wf, wr]
    pc = pl.pallas_call(
        functools.partial(_proj_body, n_in == 2),
        grid=(NB,),
        in_specs=in_specs,
        out_specs=pl.BlockSpec((T, B, W2), lambda i: (i, 0, 0)),
        out_shape=jax.ShapeDtypeStruct((S, B, W2), jnp.float32),
        compiler_params=_ARB,
    )(*operands)
    out_specs = [
        pl.BlockSpec((T, B, H), lambda i: (i, 0, 0)),
        pl.BlockSpec((T, B, H), lambda i: (NB - 1 - i, 0, 0)),
    ]
    out_shape = [jax.ShapeDtypeStruct((S, B, H), jnp.float32)] * 2
    if emit_flipped:
        out_specs += [
            pl.BlockSpec((T, B, H), lambda i: (NB - 1 - i, 0, 0)),
            pl.BlockSpec((T, B, H), lambda i: (i, 0, 0)),
        ]
        out_shape += [jax.ShapeDtypeStruct((S, B, H), jnp.float32)] * 2
    return pl.pallas_call(
        functools.partial(_rec_body, emit_flipped),
        grid=(NB,),
        in_specs=[
            pl.BlockSpec((T, B, W2), lambda i: (i, 0, 0)),
            pl.BlockSpec((1, W2), lambda i: (0, 0)),
            pl.BlockSpec((1, W2), lambda i: (0, 0)),
            pl.BlockSpec((2 * H, W2), lambda i: (0, 0)),
        ],
        out_specs=out_specs,
        out_shape=out_shape,
        scratch_shapes=[
            pltpu.VMEM((B, 2 * H), jnp.float32),
            pltpu.VMEM((B, 2 * H), jnp.float32),
        ],
        compiler_params=_ARB,
    )(pc, bihb, bhhb, wbd)


def _score_body(f_ref, r_ref, w_ref, b_ref, s3_ref):
    xc = jnp.concatenate([f_ref[...], r_ref[...]], axis=-1).reshape(T * B, D)
    s = jnp.dot(xc, w_ref[...])
    s = jax.nn.sigmoid(s + b_ref[0, 0])
    s3_ref[...] = s.reshape(T, B, D)


def _topk_body(sbt_ref, stb_ref, x_ref, pe_ref, feat_ref, posg_ref, loss_ref):
    b = pl.program_id(0)
    s_row = sbt_ref[...].reshape(1, S)
    stb = stb_ref[...]
    bmask = lax.broadcasted_iota(jnp.int32, (1, B), 1) == b
    s_col = jnp.sum(jnp.where(bmask, stb, 0.0), axis=1, keepdims=True)
    sp = lax.broadcast_in_dim(s_col, (S, S), (0, 1))
    sl = lax.broadcast_in_dim(s_row, (S, S), (0, 1))
    pidx = lax.broadcasted_iota(jnp.int32, (S, S), 0)
    iidx = lax.broadcasted_iota(jnp.int32, (S, S), 1)
    ahead = (sp > sl) | ((sp == sl) & (pidx < iidx))
    rank = jnp.sum(ahead.astype(jnp.int32), axis=0, keepdims=True)
    oh = (lax.broadcasted_iota(jnp.int32, (K, S), 0) == rank).astype(jnp.float32)
    xb = x_ref[...].reshape(S, D)
    pe = pe_ref[...].reshape(S, D)
    gx = lax.dot(oh, xb, precision=lax.Precision.HIGHEST)
    gp = lax.dot(oh, pe, precision=lax.Precision.HIGHEST)
    feat_ref[...] = jnp.concatenate(
        [gx.reshape(1, 1, K, D), gp.reshape(1, 1, K, D)], axis=1)
    posg_ref[...] = gp.reshape(1, K, D)

    mu = jnp.mean(s_row)
    dv = s_row - mu
    std = jnp.sqrt(jnp.sum(dv * dv) / (S - 1))

    @pl.when(b == 0)
    def _init():
        loss_ref[...] = jnp.zeros_like(loss_ref)

    loss_ref[...] += std * (1.0 / B)


def kernel(x, pos_emb, W_ih_l0, W_hh_l0, b_ih_l0, b_hh_l0,
           W_ih_l0r, W_hh_l0r, b_ih_l0r, b_hh_l0r,
           W_ih_l1, W_hh_l1, b_ih_l1, b_hh_l1,
           W_ih_l1r, W_hh_l1r, b_ih_l1r, b_hh_l1r,
           lin_w, lin_b):
    f32 = jnp.float32
    xt = jnp.swapaxes(x, 0, 1)

    def stretch(w_t, off):
        din = w_t.shape[0]
        out = jnp.zeros((din, W2), f32)
        for k in range(4):
            out = out.at[:, 128 * k + off:128 * k + off + H].set(
                w_t[:, H * k:H * (k + 1)])
        return out

    def stretch_b(b_f, b_r):
        out = jnp.zeros((1, W2), f32)
        for k in range(4):
            out = out.at[0, 128 * k:128 * k + H].set(b_f[H * k:H * (k + 1)])
            out = out.at[0, 128 * k + H:128 * (k + 1)].set(b_r[H * k:H * (k + 1)])
        return out

    def blockdiag(whh_f_t, whh_r_t):
        out = jnp.zeros((2 * H, W2), f32)
        out = out.at[0:H, :].set(stretch(whh_f_t, 0)[:, :])
        out = out.at[H:2 * H, :].set(stretch(whh_r_t, H)[:, :])
        return out

    def prep(W_ih_f, W_hh_f, b_ih_f, b_hh_f, W_ih_r, W_hh_r, b_ih_r, b_hh_r):
        return (stretch(W_ih_f.T.astype(f32), 0),
                stretch(W_ih_r.T.astype(f32), H),
                stretch_b(b_ih_f, b_ih_r),
                stretch_b(b_hh_f, b_hh_r),
                blockdiag(W_hh_f.T.astype(f32), W_hh_r.T.astype(f32)))

    args0 = prep(W_ih_l0, W_hh_l0, b_ih_l0, b_hh_l0,
                 W_ih_l0r, W_hh_l0r, b_ih_l0r, b_hh_l0r)
    args1 = prep(W_ih_l1, W_hh_l1, b_ih_l1, b_hh_l1,
                 W_ih_l1r, W_hh_l1r, b_ih_l1r, b_hh_l1r)

    xtf = jnp.flip(xt, 0)
    of0, or0, off0, orf0 = _bilstm_layer(
        [xt], [xtf], D, args0, emit_flipped=True)
    of1, or1 = _bilstm_layer(
        [of0, or0], [off0, orf0], D, args1, emit_flipped=False)

    w_pad = jnp.pad(lin_w.T, ((0, 0), (0, D - 1)))
    lb = lin_b.reshape(1, 1)
    s3 = pl.pallas_call(
        _score_body,
        grid=(NB,),
        in_specs=[
            pl.BlockSpec((T, B, H), lambda i: (i, 0, 0)),
            pl.BlockSpec((T, B, H), lambda i: (i, 0, 0)),
            pl.BlockSpec((D, D), lambda i: (0, 0)),
            pl.BlockSpec((1, 1), lambda i: (0, 0)),
        ],
        out_specs=pl.BlockSpec((T, B, D), lambda i: (i, 0, 0)),
        out_shape=jax.ShapeDtypeStruct((S, B, D), jnp.float32),
        compiler_params=_ARB,
    )(of1, or1, w_pad, lb)

    stb = s3[:, :, 0]
    sbt = jnp.swapaxes(stb, 0, 1)
    sbt3 = sbt[:, None, :]

    feat, posg, loss = pl.pallas_call(
        _topk_body,
        grid=(B,),
        in_specs=[
            pl.BlockSpec((1, 1, S), lambda b: (b, 0, 0)),
            pl.BlockSpec((S, B), lambda b: (0, 0)),
            pl.BlockSpec((1, S, D), lambda b: (b, 0, 0)),
            pl.BlockSpec((1, S, D), lambda b: (0, 0, 0)),
        ],
        out_specs=[
            pl.BlockSpec((1, 2, K, D), lambda b: (b, 0, 0, 0)),
            pl.BlockSpec((1, K, D), lambda b: (b, 0, 0)),
            pl.BlockSpec((1, 1), lambda b: (0, 0)),
        ],
        out_shape=[
            jax.ShapeDtypeStruct((B, 2, K, D), jnp.float32),
            jax.ShapeDtypeStruct((B, K, D), jnp.float32),
            jax.ShapeDtypeStruct((1, 1), jnp.float32),
        ],
        compiler_params=_ARB,
    )(sbt3, stb, x, pos_emb)

    score = sbt[:, :, None]
    return feat, posg, loss[0, 0], score


# speedup vs baseline: 1.0175x; 1.0175x over previous
import functools

import jax
import jax.numpy as jnp
from jax import lax
from jax.experimental import pallas as pl
from jax.experimental.pallas import tpu as pltpu
from jax.experimental.pallas import tpu_sc as plsc

B = 32
S = 1024
D = 128
H = 64
G = 4 * H
K = 256
NB = 8
T = S // NB

_ARB = pltpu.CompilerParams(dimension_semantics=("arbitrary",))


W2 = 8 * H


def _proj_body(two_stream, *refs):
    if two_stream:
        (xfa, xfb, xra, xrb, wf, wr, pc_ref) = refs
        xf = jnp.concatenate([xfa[...], xfb[...]], axis=-1)
        xr = jnp.concatenate([xra[...], xrb[...]], axis=-1)
    else:
        (xfa, xra, wf, wr, pc_ref) = refs
        xf = xfa[...]
        xr = xra[...]
    din = xf.shape[-1]
    pf = jnp.dot(xf.reshape(T * B, din), wf[...]).reshape(T, B, W2)
    pr = jnp.dot(xr.reshape(T * B, din), wr[...]).reshape(T, B, W2)
    pc_ref[...] = pf + pr


def _rec_body(emit_flipped, pc_ref, bihb, bhhb, wbd, *refs):
    if emit_flipped:
        of_ref, or_ref, off_ref, orf_ref, h_s, c_s = refs
    else:
        of_ref, or_ref, h_s, c_s = refs
    i = pl.program_id(0)

    @pl.when(i == 0)
    def _init():
        h_s[...] = jnp.zeros_like(h_s)
        c_s[...] = jnp.zeros_like(c_s)

    wbd_v = wbd[...]
    bihb_v = bihb[...]
    bhhb_v = bhhb[...]

    def body(t, carry):
        h, c = carry
        g = pc_ref[t] + jnp.dot(h, wbd_v)
        g = g + bihb_v
        g = g + bhhb_v
        ii = g[:, 0:2 * H]
        ff = g[:, 2 * H:4 * H]
        gg = g[:, 4 * H:6 * H]
        oo = g[:, 6 * H:8 * H]
        c2 = jax.nn.sigmoid(ff) * c + jax.nn.sigmoid(ii) * jnp.tanh(gg)
        h2 = jax.nn.sigmoid(oo) * jnp.tanh(c2)
        of_ref[t] = h2[:, 0:H]
        or_ref[T - 1 - t] = h2[:, H:2 * H]
        if emit_flipped:
            off_ref[T - 1 - t] = h2[:, 0:H]
            orf_ref[t] = h2[:, H:2 * H]
        return h2, c2

    h, c = lax.fori_loop(0, T, body, (h_s[...], c_s[...]))
    h_s[...] = h
    c_s[...] = c


def _bilstm_layer(fwd_arrs, rev_arrs, din, args, emit_flipped):
    n_in = len(fwd_arrs)
    w = din // n_in
    in_specs = ([pl.BlockSpec((T, B, w), lambda i: (i, 0, 0))
                 for _ in range(2 * n_in)])
    operands = list(fwd_arrs) + list(rev_arrs)
    wf, wr, bihb, bhhb, wbd = args
    in_specs += [
        pl.BlockSpec((din, W2), lambda i: (0, 0)),
        pl.BlockSpec((din, W2), lambda i: (0, 0)),
    ]
    operands += [wf, wr]
    pc = pl.pallas_call(
        functools.partial(_proj_body, n_in == 2),
        grid=(NB,),
        in_specs=in_specs,
        out_specs=pl.BlockSpec((T, B, W2), lambda i: (i, 0, 0)),
        out_shape=jax.ShapeDtypeStruct((S, B, W2), jnp.float32),
        compiler_params=_ARB,
    )(*operands)
    out_specs = [
        pl.BlockSpec((T, B, H), lambda i: (i, 0, 0)),
        pl.BlockSpec((T, B, H), lambda i: (NB - 1 - i, 0, 0)),
    ]
    out_shape = [jax.ShapeDtypeStruct((S, B, H), jnp.float32)] * 2
    if emit_flipped:
        out_specs += [
            pl.BlockSpec((T, B, H), lambda i: (NB - 1 - i, 0, 0)),
            pl.BlockSpec((T, B, H), lambda i: (i, 0, 0)),
        ]
        out_shape += [jax.ShapeDtypeStruct((S, B, H), jnp.float32)] * 2
    return pl.pallas_call(
        functools.partial(_rec_body, emit_flipped),
        grid=(NB,),
        in_specs=[
            pl.BlockSpec((T, B, W2), lambda i: (i, 0, 0)),
            pl.BlockSpec((1, W2), lambda i: (0, 0)),
            pl.BlockSpec((1, W2), lambda i: (0, 0)),
            pl.BlockSpec((2 * H, W2), lambda i: (0, 0)),
        ],
        out_specs=out_specs,
        out_shape=out_shape,
        scratch_shapes=[
            pltpu.VMEM((B, 2 * H), jnp.float32),
            pltpu.VMEM((B, 2 * H), jnp.float32),
        ],
        compiler_params=_ARB,
    )(pc, bihb, bhhb, wbd)


def _score_body(f_ref, r_ref, w_ref, b_ref, s3_ref):
    xc = jnp.concatenate([f_ref[...], r_ref[...]], axis=-1).reshape(T * B, D)
    s = jnp.dot(xc, w_ref[...])
    s = jax.nn.sigmoid(s + b_ref[0, 0])
    s3_ref[...] = s.reshape(T, B, D)


def _rank_body(sbt_ref, stb_ref, idxl_ref, idxg_ref, loss_ref):
    b = pl.program_id(0)
    s_row = sbt_ref[...].reshape(1, S)
    stb = stb_ref[...]
    bmask = lax.broadcasted_iota(jnp.int32, (1, B), 1) == b
    s_col = jnp.sum(jnp.where(bmask, stb, 0.0), axis=1, keepdims=True)
    sp = lax.broadcast_in_dim(s_col, (S, S), (0, 1))
    sl = lax.broadcast_in_dim(s_row, (S, S), (0, 1))
    pidx = lax.broadcasted_iota(jnp.int32, (S, S), 0)
    iidx = lax.broadcasted_iota(jnp.int32, (S, S), 1)
    ahead = (sp > sl) | ((sp == sl) & (pidx < iidx))
    rank = jnp.sum(ahead.astype(jnp.int32), axis=0, keepdims=True)
    oh = (lax.broadcasted_iota(jnp.int32, (K, S), 0) == rank).astype(jnp.int32)
    iol = lax.broadcasted_iota(jnp.int32, (K, S), 1)
    idxc = jnp.sum(oh * iol, axis=1, keepdims=True)
    idxl_ref[...] = idxc.reshape(1, K, 1)
    idxg_ref[...] = (idxc + b * S).reshape(1, K, 1)

    mu = jnp.mean(s_row)
    dv = s_row - mu
    std = jnp.sqrt(jnp.sum(dv * dv) / (S - 1))

    @pl.when(b == 0)
    def _init():
        loss_ref[...] = jnp.zeros_like(loss_ref)

    loss_ref[...] += std * (1.0 / B)


_NCHUNK = 2
_CW = K // _NCHUNK


def _sc_gather(idxl_flat, idxg_flat, x2, pe2):
    mesh = plsc.VectorSubcoreMesh(core_axis_name="c", subcore_axis_name="s")

    @functools.partial(
        pl.kernel, mesh=mesh,
        out_type=[jax.ShapeDtypeStruct((B * K, D), jnp.float32),
                  jax.ShapeDtypeStruct((B * K, D), jnp.float32)],
        scratch_types=[
            pltpu.VMEM((_CW,), jnp.int32),
            pltpu.VMEM((_CW,), jnp.int32),
            pltpu.VMEM((_CW,), jnp.int32),
            pltpu.VMEM((_CW,), jnp.int32),
            pltpu.VMEM((_CW, D), jnp.float32),
            pltpu.VMEM((_CW, D), jnp.float32),
            pltpu.VMEM((_CW, D), jnp.float32),
            pltpu.VMEM((_CW, D), jnp.float32),
            pltpu.SemaphoreType.DMA,
        ],
    )
    def k(idxl_hbm, idxg_hbm, x_hbm, pe_hbm, gx_hbm, gp_hbm,
          idx_a, idx_b, gidx_a, gidx_b, xr_a, xr_b, pr_a, pr_b, sem):
        b = lax.axis_index("s") * 2 + lax.axis_index("c")
        pltpu.sync_copy(idxl_hbm.at[pl.ds(b * K, _CW)], idx_a)
        pltpu.sync_copy(idxl_hbm.at[pl.ds(b * K + _CW, _CW)], idx_b)
        pltpu.sync_copy(idxg_hbm.at[pl.ds(b * K, _CW)], gidx_a)
        pltpu.sync_copy(idxg_hbm.at[pl.ds(b * K + _CW, _CW)], gidx_b)
        copies = [
            pltpu.async_copy(x_hbm.at[gidx_a], xr_a, sem),
            pltpu.async_copy(x_hbm.at[gidx_b], xr_b, sem),
            pltpu.async_copy(pe_hbm.at[idx_a], pr_a, sem),
            pltpu.async_copy(pe_hbm.at[idx_b], pr_b, sem),
        ]
        for cp in copies:
            cp.wait()
        pltpu.sync_copy(xr_a, gx_hbm.at[pl.ds(b * K, _CW)])
        pltpu.sync_copy(xr_b, gx_hbm.at[pl.ds(b * K + _CW, _CW)])
        pltpu.sync_copy(pr_a, gp_hbm.at[pl.ds(b * K, _CW)])
        pltpu.sync_copy(pr_b, gp_hbm.at[pl.ds(b * K + _CW, _CW)])

    return k(idxl_flat, idxg_flat, x2, pe2)


def kernel(x, pos_emb, W_ih_l0, W_hh_l0, b_ih_l0, b_hh_l0,
           W_ih_l0r, W_hh_l0r, b_ih_l0r, b_hh_l0r,
           W_ih_l1, W_hh_l1, b_ih_l1, b_hh_l1,
           W_ih_l1r, W_hh_l1r, b_ih_l1r, b_hh_l1r,
           lin_w, lin_b):
    f32 = jnp.float32
    xt = jnp.swapaxes(x, 0, 1)

    def stretch(w_t, off):
        din = w_t.shape[0]
        out = jnp.zeros((din, W2), f32)
        for k in range(4):
            out = out.at[:, 128 * k + off:128 * k + off + H].set(
                w_t[:, H * k:H * (k + 1)])
        return out

    def stretch_b(b_f, b_r):
        out = jnp.zeros((1, W2), f32)
        for k in range(4):
            out = out.at[0, 128 * k:128 * k + H].set(b_f[H * k:H * (k + 1)])
            out = out.at[0, 128 * k + H:128 * (k + 1)].set(b_r[H * k:H * (k + 1)])
        return out

    def blockdiag(whh_f_t, whh_r_t):
        out = jnp.zeros((2 * H, W2), f32)
        out = out.at[0:H, :].set(stretch(whh_f_t, 0)[:, :])
        out = out.at[H:2 * H, :].set(stretch(whh_r_t, H)[:, :])
        return out

    def prep(W_ih_f, W_hh_f, b_ih_f, b_hh_f, W_ih_r, W_hh_r, b_ih_r, b_hh_r):
        return (stretch(W_ih_f.T.astype(f32), 0),
                stretch(W_ih_r.T.astype(f32), H),
                stretch_b(b_ih_f, b_ih_r),
                stretch_b(b_hh_f, b_hh_r),
                blockdiag(W_hh_f.T.astype(f32), W_hh_r.T.astype(f32)))

    args0 = prep(W_ih_l0, W_hh_l0, b_ih_l0, b_hh_l0,
                 W_ih_l0r, W_hh_l0r, b_ih_l0r, b_hh_l0r)
    args1 = prep(W_ih_l1, W_hh_l1, b_ih_l1, b_hh_l1,
                 W_ih_l1r, W_hh_l1r, b_ih_l1r, b_hh_l1r)

    xtf = jnp.flip(xt, 0)
    of0, or0, off0, orf0 = _bilstm_layer(
        [xt], [xtf], D, args0, emit_flipped=True)
    of1, or1 = _bilstm_layer(
        [of0, or0], [off0, orf0], D, args1, emit_flipped=False)

    w_pad = jnp.pad(lin_w.T, ((0, 0), (0, D - 1)))
    lb = lin_b.reshape(1, 1)
    s3 = pl.pallas_call(
        _score_body,
        grid=(NB,),
        in_specs=[
            pl.BlockSpec((T, B, H), lambda i: (i, 0, 0)),
            pl.BlockSpec((T, B, H), lambda i: (i, 0, 0)),
            pl.BlockSpec((D, D), lambda i: (0, 0)),
            pl.BlockSpec((1, 1), lambda i: (0, 0)),
        ],
        out_specs=pl.BlockSpec((T, B, D), lambda i: (i, 0, 0)),
        out_shape=jax.ShapeDtypeStruct((S, B, D), jnp.float32),
        compiler_params=_ARB,
    )(of1, or1, w_pad, lb)

    stb = s3[:, :, 0]
    sbt = jnp.swapaxes(stb, 0, 1)
    sbt3 = sbt[:, None, :]

    idxl, idxg, loss = pl.pallas_call(
        _rank_body,
        grid=(B,),
        in_specs=[
            pl.BlockSpec((1, 1, S), lambda b: (b, 0, 0)),
            pl.BlockSpec((S, B), lambda b: (0, 0)),
        ],
        out_specs=[
            pl.BlockSpec((1, K, 1), lambda b: (b, 0, 0)),
            pl.BlockSpec((1, K, 1), lambda b: (b, 0, 0)),
            pl.BlockSpec((1, 1), lambda b: (0, 0)),
        ],
        out_shape=[
            jax.ShapeDtypeStruct((B, K, 1), jnp.int32),
            jax.ShapeDtypeStruct((B, K, 1), jnp.int32),
            jax.ShapeDtypeStruct((1, 1), jnp.float32),
        ],
        compiler_params=_ARB,
    )(sbt3, stb)

    gx, gp = _sc_gather(idxl.reshape(B * K), idxg.reshape(B * K),
                        x.reshape(B * S, D),
                        pos_emb.reshape(S, D))
    gx3 = gx.reshape(B, K, D)
    gp3 = gp.reshape(B, K, D)
    feat = jnp.stack([gx3, gp3], axis=1)

    score = sbt[:, :, None]
    return feat, gp3, loss[0, 0], score


# speedup vs baseline: 1.1528x; 1.1329x over previous
import functools

import jax
import jax.numpy as jnp
from jax import lax
from jax.experimental import pallas as pl
from jax.experimental.pallas import tpu as pltpu
from jax.experimental.pallas import tpu_sc as plsc

B = 32
S = 1024
D = 128
H = 64
G = 4 * H
K = 256
NB = 8
T = S // NB

_ARB = pltpu.CompilerParams(dimension_semantics=("arbitrary",))


W2 = 8 * H


def _proj_body(two_stream, *refs):
    if two_stream:
        (xfa, xfb, xra, xrb, wf, wr, pc_ref) = refs
        xf = jnp.concatenate([xfa[...], xfb[...]], axis=-1)
        xr = jnp.concatenate([xra[...], xrb[...]], axis=-1)
    else:
        (xfa, xra, wf, wr, pc_ref) = refs
        xf = xfa[...]
        xr = xra[...]
    din = xf.shape[-1]
    pf = jnp.dot(xf.reshape(T * B, din), wf[...]).reshape(T, B, W2)
    pr = jnp.dot(xr.reshape(T * B, din), wr[...]).reshape(T, B, W2)
    pc_ref[...] = pf + pr


def _rec_body(emit_flipped, pc_ref, bihb, bhhb, wbd, *refs):
    if emit_flipped:
        of_ref, or_ref, off_ref, orf_ref, h_s, c_s = refs
    else:
        of_ref, or_ref, h_s, c_s = refs
    i = pl.program_id(0)

    @pl.when(i == 0)
    def _init():
        h_s[...] = jnp.zeros_like(h_s)
        c_s[...] = jnp.zeros_like(c_s)

    wbd_v = wbd[...]
    bihb_v = bihb[...]
    bhhb_v = bhhb[...]

    def body(t, carry):
        h, c = carry
        g = pc_ref[t] + jnp.dot(h, wbd_v)
        g = g + bihb_v
        g = g + bhhb_v
        ii = g[:, 0:2 * H]
        ff = g[:, 2 * H:4 * H]
        gg = g[:, 4 * H:6 * H]
        oo = g[:, 6 * H:8 * H]
        c2 = jax.nn.sigmoid(ff) * c + jax.nn.sigmoid(ii) * jnp.tanh(gg)
        h2 = jax.nn.sigmoid(oo) * jnp.tanh(c2)
        of_ref[t] = h2[:, 0:H]
        or_ref[T - 1 - t] = h2[:, H:2 * H]
        if emit_flipped:
            off_ref[T - 1 - t] = h2[:, 0:H]
            orf_ref[t] = h2[:, H:2 * H]
        return h2, c2

    h, c = lax.fori_loop(0, T, body, (h_s[...], c_s[...]), unroll=2)
    h_s[...] = h
    c_s[...] = c


def _bilstm_layer(fwd_arrs, rev_arrs, din, args, emit_flipped):
    n_in = len(fwd_arrs)
    w = din // n_in
    in_specs = ([pl.BlockSpec((T, B, w), lambda i: (i, 0, 0))
                 for _ in range(2 * n_in)])
    operands = list(fwd_arrs) + list(rev_arrs)
    wf, wr, bihb, bhhb, wbd = args
    in_specs += [
        pl.BlockSpec((din, W2), lambda i: (0, 0)),
        pl.BlockSpec((din, W2), lambda i: (0, 0)),
    ]
    operands += [wf, wr]
    pc = pl.pallas_call(
        functools.partial(_proj_body, n_in == 2),
        grid=(NB,),
        in_specs=in_specs,
        out_specs=pl.BlockSpec((T, B, W2), lambda i: (i, 0, 0)),
        out_shape=jax.ShapeDtypeStruct((S, B, W2), jnp.float32),
        compiler_params=_ARB,
    )(*operands)
    out_specs = [
        pl.BlockSpec((T, B, H), lambda i: (i, 0, 0)),
        pl.BlockSpec((T, B, H), lambda i: (NB - 1 - i, 0, 0)),
    ]
    out_shape = [jax.ShapeDtypeStruct((S, B, H), jnp.float32)] * 2
    if emit_flipped:
        out_specs += [
            pl.BlockSpec((T, B, H), lambda i: (NB - 1 - i, 0, 0)),
            pl.BlockSpec((T, B, H), lambda i: (i, 0, 0)),
        ]
        out_shape += [jax.ShapeDtypeStruct((S, B, H), jnp.float32)] * 2
    return pl.pallas_call(
        functools.partial(_rec_body, emit_flipped),
        grid=(NB,),
        in_specs=[
            pl.BlockSpec((T, B, W2), lambda i: (i, 0, 0)),
            pl.BlockSpec((1, W2), lambda i: (0, 0)),
            pl.BlockSpec((1, W2), lambda i: (0, 0)),
            pl.BlockSpec((2 * H, W2), lambda i: (0, 0)),
        ],
        out_specs=out_specs,
        out_shape=out_shape,
        scratch_shapes=[
            pltpu.VMEM((B, 2 * H), jnp.float32),
            pltpu.VMEM((B, 2 * H), jnp.float32),
        ],
        compiler_params=_ARB,
    )(pc, bihb, bhhb, wbd)


def _score_body(f_ref, r_ref, w_ref, b_ref, s3_ref):
    xc = jnp.concatenate([f_ref[...], r_ref[...]], axis=-1).reshape(T * B, D)
    s = jnp.dot(xc, w_ref[...])
    s = jax.nn.sigmoid(s + b_ref[0, 0])
    s3_ref[...] = s.reshape(T, B, D)


def _rank_body(sbt_ref, stb_ref, idxl_ref, idxg_ref, loss_ref):
    b = pl.program_id(0)
    s_row = sbt_ref[...].reshape(1, S)
    stb = stb_ref[...]
    bmask = lax.broadcasted_iota(jnp.int32, (1, B), 1) == b
    s_col = jnp.sum(jnp.where(bmask, stb, 0.0), axis=1, keepdims=True)
    sp = lax.broadcast_in_dim(s_col, (S, S), (0, 1))
    sl = lax.broadcast_in_dim(s_row, (S, S), (0, 1))
    pidx = lax.broadcasted_iota(jnp.int32, (S, S), 0)
    iidx = lax.broadcasted_iota(jnp.int32, (S, S), 1)
    ahead = (sp > sl) | ((sp == sl) & (pidx < iidx))
    rank = jnp.sum(ahead.astype(jnp.int32), axis=0, keepdims=True)
    oh = (lax.broadcasted_iota(jnp.int32, (K, S), 0) == rank).astype(jnp.int32)
    iol = lax.broadcasted_iota(jnp.int32, (K, S), 1)
    idxc = jnp.sum(oh * iol, axis=1, keepdims=True)
    idxl_ref[...] = idxc.reshape(1, K, 1)
    idxg_ref[...] = (idxc + b * S).reshape(1, K, 1)

    mu = jnp.mean(s_row)
    dv = s_row - mu
    std = jnp.sqrt(jnp.sum(dv * dv) / (S - 1))

    @pl.when(b == 0)
    def _init():
        loss_ref[...] = jnp.zeros_like(loss_ref)

    loss_ref[...] += std * (1.0 / B)


_NCHUNK = 2
_CW = K // _NCHUNK


def _sc_gather(idxl_flat, idxg_flat, x2, pe2):
    mesh = plsc.VectorSubcoreMesh(core_axis_name="c", subcore_axis_name="s")

    @functools.partial(
        pl.kernel, mesh=mesh,
        out_type=[jax.ShapeDtypeStruct((B * K, D), jnp.float32),
                  jax.ShapeDtypeStruct((B * K, D), jnp.float32)],
        scratch_types=[
            pltpu.VMEM((_CW,), jnp.int32),
            pltpu.VMEM((_CW,), jnp.int32),
            pltpu.VMEM((_CW,), jnp.int32),
            pltpu.VMEM((_CW,), jnp.int32),
            pltpu.VMEM((_CW, D), jnp.float32),
            pltpu.VMEM((_CW, D), jnp.float32),
            pltpu.VMEM((_CW, D), jnp.float32),
            pltpu.VMEM((_CW, D), jnp.float32),
            pltpu.SemaphoreType.DMA,
        ],
    )
    def k(idxl_hbm, idxg_hbm, x_hbm, pe_hbm, gx_hbm, gp_hbm,
          idx_a, idx_b, gidx_a, gidx_b, xr_a, xr_b, pr_a, pr_b, sem):
        b = lax.axis_index("s") * 2 + lax.axis_index("c")
        pltpu.sync_copy(idxl_hbm.at[pl.ds(b * K, _CW)], idx_a)
        pltpu.sync_copy(idxl_hbm.at[pl.ds(b * K + _CW, _CW)], idx_b)
        pltpu.sync_copy(idxg_hbm.at[pl.ds(b * K, _CW)], gidx_a)
        pltpu.sync_copy(idxg_hbm.at[pl.ds(b * K + _CW, _CW)], gidx_b)
        copies = [
            pltpu.async_copy(x_hbm.at[gidx_a], xr_a, sem),
            pltpu.async_copy(x_hbm.at[gidx_b], xr_b, sem),
            pltpu.async_copy(pe_hbm.at[idx_a], pr_a, sem),
            pltpu.async_copy(pe_hbm.at[idx_b], pr_b, sem),
        ]
        for cp in copies:
            cp.wait()
        pltpu.sync_copy(xr_a, gx_hbm.at[pl.ds(b * K, _CW)])
        pltpu.sync_copy(xr_b, gx_hbm.at[pl.ds(b * K + _CW, _CW)])
        pltpu.sync_copy(pr_a, gp_hbm.at[pl.ds(b * K, _CW)])
        pltpu.sync_copy(pr_b, gp_hbm.at[pl.ds(b * K + _CW, _CW)])

    return k(idxl_flat, idxg_flat, x2, pe2)


def kernel(x, pos_emb, W_ih_l0, W_hh_l0, b_ih_l0, b_hh_l0,
           W_ih_l0r, W_hh_l0r, b_ih_l0r, b_hh_l0r,
           W_ih_l1, W_hh_l1, b_ih_l1, b_hh_l1,
           W_ih_l1r, W_hh_l1r, b_ih_l1r, b_hh_l1r,
           lin_w, lin_b):
    f32 = jnp.float32
    xt = jnp.swapaxes(x, 0, 1)

    def stretch(w_t, off):
        din = w_t.shape[0]
        out = jnp.zeros((din, W2), f32)
        for k in range(4):
            out = out.at[:, 128 * k + off:128 * k + off + H].set(
                w_t[:, H * k:H * (k + 1)])
        return out

    def stretch_b(b_f, b_r):
        out = jnp.zeros((1, W2), f32)
        for k in range(4):
            out = out.at[0, 128 * k:128 * k + H].set(b_f[H * k:H * (k + 1)])
            out = out.at[0, 128 * k + H:128 * (k + 1)].set(b_r[H * k:H * (k + 1)])
        return out

    def blockdiag(whh_f_t, whh_r_t):
        out = jnp.zeros((2 * H, W2), f32)
        out = out.at[0:H, :].set(stretch(whh_f_t, 0)[:, :])
        out = out.at[H:2 * H, :].set(stretch(whh_r_t, H)[:, :])
        return out

    def prep(W_ih_f, W_hh_f, b_ih_f, b_hh_f, W_ih_r, W_hh_r, b_ih_r, b_hh_r):
        return (stretch(W_ih_f.T.astype(f32), 0),
                stretch(W_ih_r.T.astype(f32), H),
                stretch_b(b_ih_f, b_ih_r),
                stretch_b(b_hh_f, b_hh_r),
                blockdiag(W_hh_f.T.astype(f32), W_hh_r.T.astype(f32)))

    args0 = prep(W_ih_l0, W_hh_l0, b_ih_l0, b_hh_l0,
                 W_ih_l0r, W_hh_l0r, b_ih_l0r, b_hh_l0r)
    args1 = prep(W_ih_l1, W_hh_l1, b_ih_l1, b_hh_l1,
                 W_ih_l1r, W_hh_l1r, b_ih_l1r, b_hh_l1r)

    xtf = jnp.flip(xt, 0)
    of0, or0, off0, orf0 = _bilstm_layer(
        [xt], [xtf], D, args0, emit_flipped=True)
    of1, or1 = _bilstm_layer(
        [of0, or0], [off0, orf0], D, args1, emit_flipped=False)

    w_pad = jnp.pad(lin_w.T, ((0, 0), (0, D - 1)))
    lb = lin_b.reshape(1, 1)
    s3 = pl.pallas_call(
        _score_body,
        grid=(NB,),
        in_specs=[
            pl.BlockSpec((T, B, H), lambda i: (i, 0, 0)),
            pl.BlockSpec((T, B, H), lambda i: (i, 0, 0)),
            pl.BlockSpec((D, D), lambda i: (0, 0)),
            pl.BlockSpec((1, 1), lambda i: (0, 0)),
        ],
        out_specs=pl.BlockSpec((T, B, D), lambda i: (i, 0, 0)),
        out_shape=jax.ShapeDtypeStruct((S, B, D), jnp.float32),
        compiler_params=_ARB,
    )(of1, or1, w_pad, lb)

    stb = s3[:, :, 0]
    sbt = jnp.swapaxes(stb, 0, 1)
    sbt3 = sbt[:, None, :]

    idxl, idxg, loss = pl.pallas_call(
        _rank_body,
        grid=(B,),
        in_specs=[
            pl.BlockSpec((1, 1, S), lambda b: (b, 0, 0)),
            pl.BlockSpec((S, B), lambda b: (0, 0)),
        ],
        out_specs=[
            pl.BlockSpec((1, K, 1), lambda b: (b, 0, 0)),
            pl.BlockSpec((1, K, 1), lambda b: (b, 0, 0)),
            pl.BlockSpec((1, 1), lambda b: (0, 0)),
        ],
        out_shape=[
            jax.ShapeDtypeStruct((B, K, 1), jnp.int32),
            jax.ShapeDtypeStruct((B, K, 1), jnp.int32),
            jax.ShapeDtypeStruct((1, 1), jnp.float32),
        ],
        compiler_params=_ARB,
    )(sbt3, stb)

    gx, gp = _sc_gather(idxl.reshape(B * K), idxg.reshape(B * K),
                        x.reshape(B * S, D),
                        pos_emb.reshape(S, D))
    gx3 = gx.reshape(B, K, D)
    gp3 = gp.reshape(B, K, D)
    feat = jnp.stack([gx3, gp3], axis=1)

    score = sbt[:, :, None]
    return feat, gp3, loss[0, 0], score


# speedup vs baseline: 1.2408x; 1.0764x over previous
import functools

import jax
import jax.numpy as jnp
from jax import lax
from jax.experimental import pallas as pl
from jax.experimental.pallas import tpu as pltpu
from jax.experimental.pallas import tpu_sc as plsc

B = 32
S = 1024
D = 128
H = 64
G = 4 * H
K = 256
NB = 8
T = S // NB

_ARB = pltpu.CompilerParams(dimension_semantics=("arbitrary",))


W2 = 8 * H


def _proj_body(two_stream, *refs):
    if two_stream:
        (xfa, xfb, xra, xrb, wf, wr, pc_ref) = refs
        xf = jnp.concatenate([xfa[...], xfb[...]], axis=-1)
        xr = jnp.concatenate([xra[...], xrb[...]], axis=-1)
    else:
        (xfa, xra, wf, wr, pc_ref) = refs
        xf = xfa[...]
        xr = xra[...]
    din = xf.shape[-1]
    pf = jnp.dot(xf.reshape(T * B, din), wf[...]).reshape(T, B, W2)
    pr = jnp.dot(xr.reshape(T * B, din), wr[...]).reshape(T, B, W2)
    pc_ref[...] = pf + pr


def _rec_body(emit_flipped, pc_ref, bihb, bhhb, wbd, *refs):
    if emit_flipped:
        of_ref, or_ref, off_ref, orf_ref, h_s, c_s = refs
    else:
        of_ref, or_ref, h_s, c_s = refs
    i = pl.program_id(0)

    @pl.when(i == 0)
    def _init():
        h_s[...] = jnp.zeros_like(h_s)
        c_s[...] = jnp.zeros_like(c_s)

    wbd_v = wbd[...]
    bihb_v = bihb[...]
    bhhb_v = bhhb[...]

    def body(t, carry):
        h, c = carry
        g = pc_ref[t] + jnp.dot(h, wbd_v)
        g = g + bihb_v
        g = g + bhhb_v
        ii = g[:, 0:2 * H]
        ff = g[:, 2 * H:4 * H]
        gg = g[:, 4 * H:6 * H]
        oo = g[:, 6 * H:8 * H]
        c2 = jax.nn.sigmoid(ff) * c + jax.nn.sigmoid(ii) * jnp.tanh(gg)
        h2 = jax.nn.sigmoid(oo) * jnp.tanh(c2)
        of_ref[t] = h2[:, 0:H]
        or_ref[T - 1 - t] = h2[:, H:2 * H]
        if emit_flipped:
            off_ref[T - 1 - t] = h2[:, 0:H]
            orf_ref[t] = h2[:, H:2 * H]
        return h2, c2

    h, c = lax.fori_loop(0, T, body, (h_s[...], c_s[...]), unroll=4)
    h_s[...] = h
    c_s[...] = c


def _bilstm_layer(fwd_arrs, rev_arrs, din, args, emit_flipped):
    n_in = len(fwd_arrs)
    w = din // n_in
    in_specs = ([pl.BlockSpec((T, B, w), lambda i: (i, 0, 0))
                 for _ in range(2 * n_in)])
    operands = list(fwd_arrs) + list(rev_arrs)
    wf, wr, bihb, bhhb, wbd = args
    in_specs += [
        pl.BlockSpec((din, W2), lambda i: (0, 0)),
        pl.BlockSpec((din, W2), lambda i: (0, 0)),
    ]
    operands += [wf, wr]
    pc = pl.pallas_call(
        functools.partial(_proj_body, n_in == 2),
        grid=(NB,),
        in_specs=in_specs,
        out_specs=pl.BlockSpec((T, B, W2), lambda i: (i, 0, 0)),
        out_shape=jax.ShapeDtypeStruct((S, B, W2), jnp.float32),
        compiler_params=_ARB,
    )(*operands)
    out_specs = [
        pl.BlockSpec((T, B, H), lambda i: (i, 0, 0)),
        pl.BlockSpec((T, B, H), lambda i: (NB - 1 - i, 0, 0)),
    ]
    out_shape = [jax.ShapeDtypeStruct((S, B, H), jnp.float32)] * 2
    if emit_flipped:
        out_specs += [
            pl.BlockSpec((T, B, H), lambda i: (NB - 1 - i, 0, 0)),
            pl.BlockSpec((T, B, H), lambda i: (i, 0, 0)),
        ]
        out_shape += [jax.ShapeDtypeStruct((S, B, H), jnp.float32)] * 2
    return pl.pallas_call(
        functools.partial(_rec_body, emit_flipped),
        grid=(NB,),
        in_specs=[
            pl.BlockSpec((T, B, W2), lambda i: (i, 0, 0)),
            pl.BlockSpec((1, W2), lambda i: (0, 0)),
            pl.BlockSpec((1, W2), lambda i: (0, 0)),
            pl.BlockSpec((2 * H, W2), lambda i: (0, 0)),
        ],
        out_specs=out_specs,
        out_shape=out_shape,
        scratch_shapes=[
            pltpu.VMEM((B, 2 * H), jnp.float32),
            pltpu.VMEM((B, 2 * H), jnp.float32),
        ],
        compiler_params=_ARB,
    )(pc, bihb, bhhb, wbd)


def _score_body(f_ref, r_ref, w_ref, b_ref, s3_ref):
    xc = jnp.concatenate([f_ref[...], r_ref[...]], axis=-1).reshape(T * B, D)
    s = jnp.dot(xc, w_ref[...])
    s = jax.nn.sigmoid(s + b_ref[0, 0])
    s3_ref[...] = s.reshape(T, B, D)


def _rank_body(sbt_ref, stb_ref, idxl_ref, idxg_ref, loss_ref):
    b = pl.program_id(0)
    s_row = sbt_ref[...].reshape(1, S)
    stb = stb_ref[...]
    bmask = lax.broadcasted_iota(jnp.int32, (1, B), 1) == b
    s_col = jnp.sum(jnp.where(bmask, stb, 0.0), axis=1, keepdims=True)
    sp = lax.broadcast_in_dim(s_col, (S, S), (0, 1))
    sl = lax.broadcast_in_dim(s_row, (S, S), (0, 1))
    pidx = lax.broadcasted_iota(jnp.int32, (S, S), 0)
    iidx = lax.broadcasted_iota(jnp.int32, (S, S), 1)
    ahead = (sp > sl) | ((sp == sl) & (pidx < iidx))
    rank = jnp.sum(ahead.astype(jnp.int32), axis=0, keepdims=True)
    oh = (lax.broadcasted_iota(jnp.int32, (K, S), 0) == rank).astype(jnp.int32)
    iol = lax.broadcasted_iota(jnp.int32, (K, S), 1)
    idxc = jnp.sum(oh * iol, axis=1, keepdims=True)
    idxl_ref[...] = idxc.reshape(1, K, 1)
    idxg_ref[...] = (idxc + b * S).reshape(1, K, 1)

    mu = jnp.mean(s_row)
    dv = s_row - mu
    std = jnp.sqrt(jnp.sum(dv * dv) / (S - 1))

    @pl.when(b == 0)
    def _init():
        loss_ref[...] = jnp.zeros_like(loss_ref)

    loss_ref[...] += std * (1.0 / B)


_NCHUNK = 2
_CW = K // _NCHUNK


def _sc_gather(idxl_flat, idxg_flat, x2, pe2):
    mesh = plsc.VectorSubcoreMesh(core_axis_name="c", subcore_axis_name="s")

    @functools.partial(
        pl.kernel, mesh=mesh,
        out_type=[jax.ShapeDtypeStruct((B * K, D), jnp.float32),
                  jax.ShapeDtypeStruct((B * K, D), jnp.float32)],
        scratch_types=[
            pltpu.VMEM((_CW,), jnp.int32),
            pltpu.VMEM((_CW,), jnp.int32),
            pltpu.VMEM((_CW,), jnp.int32),
            pltpu.VMEM((_CW,), jnp.int32),
            pltpu.VMEM((_CW, D), jnp.float32),
            pltpu.VMEM((_CW, D), jnp.float32),
            pltpu.VMEM((_CW, D), jnp.float32),
            pltpu.VMEM((_CW, D), jnp.float32),
            pltpu.SemaphoreType.DMA,
        ],
    )
    def k(idxl_hbm, idxg_hbm, x_hbm, pe_hbm, gx_hbm, gp_hbm,
          idx_a, idx_b, gidx_a, gidx_b, xr_a, xr_b, pr_a, pr_b, sem):
        b = lax.axis_index("s") * 2 + lax.axis_index("c")
        pltpu.sync_copy(idxl_hbm.at[pl.ds(b * K, _CW)], idx_a)
        pltpu.sync_copy(idxl_hbm.at[pl.ds(b * K + _CW, _CW)], idx_b)
        pltpu.sync_copy(idxg_hbm.at[pl.ds(b * K, _CW)], gidx_a)
        pltpu.sync_copy(idxg_hbm.at[pl.ds(b * K + _CW, _CW)], gidx_b)
        copies = [
            pltpu.async_copy(x_hbm.at[gidx_a], xr_a, sem),
            pltpu.async_copy(x_hbm.at[gidx_b], xr_b, sem),
            pltpu.async_copy(pe_hbm.at[idx_a], pr_a, sem),
            pltpu.async_copy(pe_hbm.at[idx_b], pr_b, sem),
        ]
        for cp in copies:
            cp.wait()
        pltpu.sync_copy(xr_a, gx_hbm.at[pl.ds(b * K, _CW)])
        pltpu.sync_copy(xr_b, gx_hbm.at[pl.ds(b * K + _CW, _CW)])
        pltpu.sync_copy(pr_a, gp_hbm.at[pl.ds(b * K, _CW)])
        pltpu.sync_copy(pr_b, gp_hbm.at[pl.ds(b * K + _CW, _CW)])

    return k(idxl_flat, idxg_flat, x2, pe2)


def kernel(x, pos_emb, W_ih_l0, W_hh_l0, b_ih_l0, b_hh_l0,
           W_ih_l0r, W_hh_l0r, b_ih_l0r, b_hh_l0r,
           W_ih_l1, W_hh_l1, b_ih_l1, b_hh_l1,
           W_ih_l1r, W_hh_l1r, b_ih_l1r, b_hh_l1r,
           lin_w, lin_b):
    f32 = jnp.float32
    xt = jnp.swapaxes(x, 0, 1)

    def stretch(w_t, off):
        din = w_t.shape[0]
        out = jnp.zeros((din, W2), f32)
        for k in range(4):
            out = out.at[:, 128 * k + off:128 * k + off + H].set(
                w_t[:, H * k:H * (k + 1)])
        return out

    def stretch_b(b_f, b_r):
        out = jnp.zeros((1, W2), f32)
        for k in range(4):
            out = out.at[0, 128 * k:128 * k + H].set(b_f[H * k:H * (k + 1)])
            out = out.at[0, 128 * k + H:128 * (k + 1)].set(b_r[H * k:H * (k + 1)])
        return out

    def blockdiag(whh_f_t, whh_r_t):
        out = jnp.zeros((2 * H, W2), f32)
        out = out.at[0:H, :].set(stretch(whh_f_t, 0)[:, :])
        out = out.at[H:2 * H, :].set(stretch(whh_r_t, H)[:, :])
        return out

    def prep(W_ih_f, W_hh_f, b_ih_f, b_hh_f, W_ih_r, W_hh_r, b_ih_r, b_hh_r):
        return (stretch(W_ih_f.T.astype(f32), 0),
                stretch(W_ih_r.T.astype(f32), H),
                stretch_b(b_ih_f, b_ih_r),
                stretch_b(b_hh_f, b_hh_r),
                blockdiag(W_hh_f.T.astype(f32), W_hh_r.T.astype(f32)))

    args0 = prep(W_ih_l0, W_hh_l0, b_ih_l0, b_hh_l0,
                 W_ih_l0r, W_hh_l0r, b_ih_l0r, b_hh_l0r)
    args1 = prep(W_ih_l1, W_hh_l1, b_ih_l1, b_hh_l1,
                 W_ih_l1r, W_hh_l1r, b_ih_l1r, b_hh_l1r)

    xtf = jnp.flip(xt, 0)
    of0, or0, off0, orf0 = _bilstm_layer(
        [xt], [xtf], D, args0, emit_flipped=True)
    of1, or1 = _bilstm_layer(
        [of0, or0], [off0, orf0], D, args1, emit_flipped=False)

    w_pad = jnp.pad(lin_w.T, ((0, 0), (0, D - 1)))
    lb = lin_b.reshape(1, 1)
    s3 = pl.pallas_call(
        _score_body,
        grid=(NB,),
        in_specs=[
            pl.BlockSpec((T, B, H), lambda i: (i, 0, 0)),
            pl.BlockSpec((T, B, H), lambda i: (i, 0, 0)),
            pl.BlockSpec((D, D), lambda i: (0, 0)),
            pl.BlockSpec((1, 1), lambda i: (0, 0)),
        ],
        out_specs=pl.BlockSpec((T, B, D), lambda i: (i, 0, 0)),
        out_shape=jax.ShapeDtypeStruct((S, B, D), jnp.float32),
        compiler_params=_ARB,
    )(of1, or1, w_pad, lb)

    stb = s3[:, :, 0]
    sbt = jnp.swapaxes(stb, 0, 1)
    sbt3 = sbt[:, None, :]

    idxl, idxg, loss = pl.pallas_call(
        _rank_body,
        grid=(B,),
        in_specs=[
            pl.BlockSpec((1, 1, S), lambda b: (b, 0, 0)),
            pl.BlockSpec((S, B), lambda b: (0, 0)),
        ],
        out_specs=[
            pl.BlockSpec((1, K, 1), lambda b: (b, 0, 0)),
            pl.BlockSpec((1, K, 1), lambda b: (b, 0, 0)),
            pl.BlockSpec((1, 1), lambda b: (0, 0)),
        ],
        out_shape=[
            jax.ShapeDtypeStruct((B, K, 1), jnp.int32),
            jax.ShapeDtypeStruct((B, K, 1), jnp.int32),
            jax.ShapeDtypeStruct((1, 1), jnp.float32),
        ],
        compiler_params=_ARB,
    )(sbt3, stb)

    gx, gp = _sc_gather(idxl.reshape(B * K), idxg.reshape(B * K),
                        x.reshape(B * S, D),
                        pos_emb.reshape(S, D))
    gx3 = gx.reshape(B, K, D)
    gp3 = gp.reshape(B, K, D)
    feat = jnp.stack([gx3, gp3], axis=1)

    score = sbt[:, :, None]
    return feat, gp3, loss[0, 0], score


# speedup vs baseline: 1.2914x; 1.0408x over previous
import functools

import jax
import jax.numpy as jnp
from jax import lax
from jax.experimental import pallas as pl
from jax.experimental.pallas import tpu as pltpu
from jax.experimental.pallas import tpu_sc as plsc

B = 32
S = 1024
D = 128
H = 64
G = 4 * H
K = 256
NB = 8
T = S // NB

_ARB = pltpu.CompilerParams(dimension_semantics=("arbitrary",))


W2 = 8 * H


def _proj_body(two_stream, *refs):
    if two_stream:
        (xfa, xfb, xra, xrb, wf, wr, pc_ref) = refs
        xf = jnp.concatenate([xfa[...], xfb[...]], axis=-1)
        xr = jnp.concatenate([xra[...], xrb[...]], axis=-1)
    else:
        (xfa, xra, wf, wr, pc_ref) = refs
        xf = xfa[...]
        xr = xra[...]
    din = xf.shape[-1]
    pf = jnp.dot(xf.reshape(T * B, din), wf[...]).reshape(T, B, W2)
    pr = jnp.dot(xr.reshape(T * B, din), wr[...]).reshape(T, B, W2)
    pc_ref[...] = pf + pr


def _rec_body(emit_flipped, pc_ref, bihb, bhhb, wbd, *refs):
    if emit_flipped:
        of_ref, or_ref, off_ref, orf_ref, h_s, c_s = refs
    else:
        of_ref, or_ref, h_s, c_s = refs
    i = pl.program_id(0)

    @pl.when(i == 0)
    def _init():
        h_s[...] = jnp.zeros_like(h_s)
        c_s[...] = jnp.zeros_like(c_s)

    wbd_v = wbd[...]
    bihb_v = bihb[...]
    bhhb_v = bhhb[...]

    def body(t, carry):
        h, c = carry
        g = pc_ref[t] + jnp.dot(h, wbd_v)
        g = g + bihb_v
        g = g + bhhb_v
        ii = g[:, 0:2 * H]
        ff = g[:, 2 * H:4 * H]
        gg = g[:, 4 * H:6 * H]
        oo = g[:, 6 * H:8 * H]
        c2 = jax.nn.sigmoid(ff) * c + jax.nn.sigmoid(ii) * jnp.tanh(gg)
        h2 = jax.nn.sigmoid(oo) * jnp.tanh(c2)
        of_ref[t] = h2[:, 0:H]
        or_ref[T - 1 - t] = h2[:, H:2 * H]
        if emit_flipped:
            off_ref[T - 1 - t] = h2[:, 0:H]
            orf_ref[t] = h2[:, H:2 * H]
        return h2, c2

    h, c = lax.fori_loop(0, T, body, (h_s[...], c_s[...]), unroll=8)
    h_s[...] = h
    c_s[...] = c


def _bilstm_layer(fwd_arrs, rev_arrs, din, args, emit_flipped):
    n_in = len(fwd_arrs)
    w = din // n_in
    in_specs = ([pl.BlockSpec((T, B, w), lambda i: (i, 0, 0))
                 for _ in range(2 * n_in)])
    operands = list(fwd_arrs) + list(rev_arrs)
    wf, wr, bihb, bhhb, wbd = args
    in_specs += [
        pl.BlockSpec((din, W2), lambda i: (0, 0)),
        pl.BlockSpec((din, W2), lambda i: (0, 0)),
    ]
    operands += [wf, wr]
    pc = pl.pallas_call(
        functools.partial(_proj_body, n_in == 2),
        grid=(NB,),
        in_specs=in_specs,
        out_specs=pl.BlockSpec((T, B, W2), lambda i: (i, 0, 0)),
        out_shape=jax.ShapeDtypeStruct((S, B, W2), jnp.float32),
        compiler_params=_ARB,
    )(*operands)
    out_specs = [
        pl.BlockSpec((T, B, H), lambda i: (i, 0, 0)),
        pl.BlockSpec((T, B, H), lambda i: (NB - 1 - i, 0, 0)),
    ]
    out_shape = [jax.ShapeDtypeStruct((S, B, H), jnp.float32)] * 2
    if emit_flipped:
        out_specs += [
            pl.BlockSpec((T, B, H), lambda i: (NB - 1 - i, 0, 0)),
            pl.BlockSpec((T, B, H), lambda i: (i, 0, 0)),
        ]
        out_shape += [jax.ShapeDtypeStruct((S, B, H), jnp.float32)] * 2
    return pl.pallas_call(
        functools.partial(_rec_body, emit_flipped),
        grid=(NB,),
        in_specs=[
            pl.BlockSpec((T, B, W2), lambda i: (i, 0, 0)),
            pl.BlockSpec((1, W2), lambda i: (0, 0)),
            pl.BlockSpec((1, W2), lambda i: (0, 0)),
            pl.BlockSpec((2 * H, W2), lambda i: (0, 0)),
        ],
        out_specs=out_specs,
        out_shape=out_shape,
        scratch_shapes=[
            pltpu.VMEM((B, 2 * H), jnp.float32),
            pltpu.VMEM((B, 2 * H), jnp.float32),
        ],
        compiler_params=_ARB,
    )(pc, bihb, bhhb, wbd)


def _score_body(f_ref, r_ref, w_ref, b_ref, s3_ref):
    xc = jnp.concatenate([f_ref[...], r_ref[...]], axis=-1).reshape(T * B, D)
    s = jnp.dot(xc, w_ref[...])
    s = jax.nn.sigmoid(s + b_ref[0, 0])
    s3_ref[...] = s.reshape(T, B, D)


def _rank_body(sbt_ref, stb_ref, idxl_ref, idxg_ref, loss_ref):
    b = pl.program_id(0)
    s_row = sbt_ref[...].reshape(1, S)
    stb = stb_ref[...]
    bmask = lax.broadcasted_iota(jnp.int32, (1, B), 1) == b
    s_col = jnp.sum(jnp.where(bmask, stb, 0.0), axis=1, keepdims=True)
    sp = lax.broadcast_in_dim(s_col, (S, S), (0, 1))
    sl = lax.broadcast_in_dim(s_row, (S, S), (0, 1))
    pidx = lax.broadcasted_iota(jnp.int32, (S, S), 0)
    iidx = lax.broadcasted_iota(jnp.int32, (S, S), 1)
    ahead = (sp > sl) | ((sp == sl) & (pidx < iidx))
    rank = jnp.sum(ahead.astype(jnp.int32), axis=0, keepdims=True)
    oh = (lax.broadcasted_iota(jnp.int32, (K, S), 0) == rank).astype(jnp.int32)
    iol = lax.broadcasted_iota(jnp.int32, (K, S), 1)
    idxc = jnp.sum(oh * iol, axis=1, keepdims=True)
    idxl_ref[...] = idxc.reshape(1, K, 1)
    idxg_ref[...] = (idxc + b * S).reshape(1, K, 1)

    mu = jnp.mean(s_row)
    dv = s_row - mu
    std = jnp.sqrt(jnp.sum(dv * dv) / (S - 1))

    @pl.when(b == 0)
    def _init():
        loss_ref[...] = jnp.zeros_like(loss_ref)

    loss_ref[...] += std * (1.0 / B)


_NCHUNK = 2
_CW = K // _NCHUNK


def _sc_gather(idxl_flat, idxg_flat, x2, pe2):
    mesh = plsc.VectorSubcoreMesh(core_axis_name="c", subcore_axis_name="s")

    @functools.partial(
        pl.kernel, mesh=mesh,
        out_type=[jax.ShapeDtypeStruct((B * K, D), jnp.float32),
                  jax.ShapeDtypeStruct((B * K, D), jnp.float32)],
        scratch_types=[
            pltpu.VMEM((_CW,), jnp.int32),
            pltpu.VMEM((_CW,), jnp.int32),
            pltpu.VMEM((_CW,), jnp.int32),
            pltpu.VMEM((_CW,), jnp.int32),
            pltpu.VMEM((_CW, D), jnp.float32),
            pltpu.VMEM((_CW, D), jnp.float32),
            pltpu.VMEM((_CW, D), jnp.float32),
            pltpu.VMEM((_CW, D), jnp.float32),
            pltpu.SemaphoreType.DMA,
        ],
    )
    def k(idxl_hbm, idxg_hbm, x_hbm, pe_hbm, gx_hbm, gp_hbm,
          idx_a, idx_b, gidx_a, gidx_b, xr_a, xr_b, pr_a, pr_b, sem):
        b = lax.axis_index("s") * 2 + lax.axis_index("c")
        pltpu.sync_copy(idxl_hbm.at[pl.ds(b * K, _CW)], idx_a)
        pltpu.sync_copy(idxl_hbm.at[pl.ds(b * K + _CW, _CW)], idx_b)
        pltpu.sync_copy(idxg_hbm.at[pl.ds(b * K, _CW)], gidx_a)
        pltpu.sync_copy(idxg_hbm.at[pl.ds(b * K + _CW, _CW)], gidx_b)
        copies = [
            pltpu.async_copy(x_hbm.at[gidx_a], xr_a, sem),
            pltpu.async_copy(x_hbm.at[gidx_b], xr_b, sem),
            pltpu.async_copy(pe_hbm.at[idx_a], pr_a, sem),
            pltpu.async_copy(pe_hbm.at[idx_b], pr_b, sem),
        ]
        for cp in copies:
            cp.wait()
        pltpu.sync_copy(xr_a, gx_hbm.at[pl.ds(b * K, _CW)])
        pltpu.sync_copy(xr_b, gx_hbm.at[pl.ds(b * K + _CW, _CW)])
        pltpu.sync_copy(pr_a, gp_hbm.at[pl.ds(b * K, _CW)])
        pltpu.sync_copy(pr_b, gp_hbm.at[pl.ds(b * K + _CW, _CW)])

    return k(idxl_flat, idxg_flat, x2, pe2)


def kernel(x, pos_emb, W_ih_l0, W_hh_l0, b_ih_l0, b_hh_l0,
           W_ih_l0r, W_hh_l0r, b_ih_l0r, b_hh_l0r,
           W_ih_l1, W_hh_l1, b_ih_l1, b_hh_l1,
           W_ih_l1r, W_hh_l1r, b_ih_l1r, b_hh_l1r,
           lin_w, lin_b):
    f32 = jnp.float32
    xt = jnp.swapaxes(x, 0, 1)

    def stretch(w_t, off):
        din = w_t.shape[0]
        out = jnp.zeros((din, W2), f32)
        for k in range(4):
            out = out.at[:, 128 * k + off:128 * k + off + H].set(
                w_t[:, H * k:H * (k + 1)])
        return out

    def stretch_b(b_f, b_r):
        out = jnp.zeros((1, W2), f32)
        for k in range(4):
            out = out.at[0, 128 * k:128 * k + H].set(b_f[H * k:H * (k + 1)])
            out = out.at[0, 128 * k + H:128 * (k + 1)].set(b_r[H * k:H * (k + 1)])
        return out

    def blockdiag(whh_f_t, whh_r_t):
        out = jnp.zeros((2 * H, W2), f32)
        out = out.at[0:H, :].set(stretch(whh_f_t, 0)[:, :])
        out = out.at[H:2 * H, :].set(stretch(whh_r_t, H)[:, :])
        return out

    def prep(W_ih_f, W_hh_f, b_ih_f, b_hh_f, W_ih_r, W_hh_r, b_ih_r, b_hh_r):
        return (stretch(W_ih_f.T.astype(f32), 0),
                stretch(W_ih_r.T.astype(f32), H),
                stretch_b(b_ih_f, b_ih_r),
                stretch_b(b_hh_f, b_hh_r),
                blockdiag(W_hh_f.T.astype(f32), W_hh_r.T.astype(f32)))

    args0 = prep(W_ih_l0, W_hh_l0, b_ih_l0, b_hh_l0,
                 W_ih_l0r, W_hh_l0r, b_ih_l0r, b_hh_l0r)
    args1 = prep(W_ih_l1, W_hh_l1, b_ih_l1, b_hh_l1,
                 W_ih_l1r, W_hh_l1r, b_ih_l1r, b_hh_l1r)

    xtf = jnp.flip(xt, 0)
    of0, or0, off0, orf0 = _bilstm_layer(
        [xt], [xtf], D, args0, emit_flipped=True)
    of1, or1 = _bilstm_layer(
        [of0, or0], [off0, orf0], D, args1, emit_flipped=False)

    w_pad = jnp.pad(lin_w.T, ((0, 0), (0, D - 1)))
    lb = lin_b.reshape(1, 1)
    s3 = pl.pallas_call(
        _score_body,
        grid=(NB,),
        in_specs=[
            pl.BlockSpec((T, B, H), lambda i: (i, 0, 0)),
            pl.BlockSpec((T, B, H), lambda i: (i, 0, 0)),
            pl.BlockSpec((D, D), lambda i: (0, 0)),
            pl.BlockSpec((1, 1), lambda i: (0, 0)),
        ],
        out_specs=pl.BlockSpec((T, B, D), lambda i: (i, 0, 0)),
        out_shape=jax.ShapeDtypeStruct((S, B, D), jnp.float32),
        compiler_params=_ARB,
    )(of1, or1, w_pad, lb)

    stb = s3[:, :, 0]
    sbt = jnp.swapaxes(stb, 0, 1)
    sbt3 = sbt[:, None, :]

    idxl, idxg, loss = pl.pallas_call(
        _rank_body,
        grid=(B,),
        in_specs=[
            pl.BlockSpec((1, 1, S), lambda b: (b, 0, 0)),
            pl.BlockSpec((S, B), lambda b: (0, 0)),
        ],
        out_specs=[
            pl.BlockSpec((1, K, 1), lambda b: (b, 0, 0)),
            pl.BlockSpec((1, K, 1), lambda b: (b, 0, 0)),
            pl.BlockSpec((1, 1), lambda b: (0, 0)),
        ],
        out_shape=[
            jax.ShapeDtypeStruct((B, K, 1), jnp.int32),
            jax.ShapeDtypeStruct((B, K, 1), jnp.int32),
            jax.ShapeDtypeStruct((1, 1), jnp.float32),
        ],
        compiler_params=_ARB,
    )(sbt3, stb)

    gx, gp = _sc_gather(idxl.reshape(B * K), idxg.reshape(B * K),
                        x.reshape(B * S, D),
                        pos_emb.reshape(S, D))
    gx3 = gx.reshape(B, K, D)
    gp3 = gp.reshape(B, K, D)
    feat = jnp.stack([gx3, gp3], axis=1)

    score = sbt[:, :, None]
    return feat, gp3, loss[0, 0], score


# speedup vs baseline: 1.3392x; 1.0370x over previous
import functools

import jax
import jax.numpy as jnp
from jax import lax
from jax.experimental import pallas as pl
from jax.experimental.pallas import tpu as pltpu
from jax.experimental.pallas import tpu_sc as plsc

B = 32
S = 1024
D = 128
H = 64
G = 4 * H
K = 256
NB = 8
T = S // NB

_ARB = pltpu.CompilerParams(dimension_semantics=("arbitrary",))


W2 = 8 * H


def _proj_body(two_stream, *refs):
    if two_stream:
        (xfa, xfb, xra, xrb, wf, wr, pc_ref) = refs
        xf = jnp.concatenate([xfa[...], xfb[...]], axis=-1)
        xr = jnp.concatenate([xra[...], xrb[...]], axis=-1)
    else:
        (xfa, xra, wf, wr, pc_ref) = refs
        xf = xfa[...]
        xr = xra[...]
    din = xf.shape[-1]
    pf = jnp.dot(xf.reshape(T * B, din), wf[...]).reshape(T, B, W2)
    pr = jnp.dot(xr.reshape(T * B, din), wr[...]).reshape(T, B, W2)
    pc_ref[...] = pf + pr


def _rec_body(emit_flipped, pc_ref, bihb, bhhb, wbd, *refs):
    if emit_flipped:
        of_ref, or_ref, off_ref, orf_ref, h_s, c_s = refs
    else:
        of_ref, or_ref, h_s, c_s = refs
    i = pl.program_id(0)

    @pl.when(i == 0)
    def _init():
        h_s[...] = jnp.zeros_like(h_s)
        c_s[...] = jnp.zeros_like(c_s)

    wbd_v = wbd[...]
    bihb_v = bihb[...]
    bhhb_v = bhhb[...]

    def body(t, carry):
        h, c = carry
        g = pc_ref[t] + jnp.dot(h, wbd_v)
        g = g + bihb_v
        g = g + bhhb_v
        ii = g[:, 0:2 * H]
        ff = g[:, 2 * H:4 * H]
        gg = g[:, 4 * H:6 * H]
        oo = g[:, 6 * H:8 * H]
        c2 = jax.nn.sigmoid(ff) * c + jax.nn.sigmoid(ii) * jnp.tanh(gg)
        h2 = jax.nn.sigmoid(oo) * jnp.tanh(c2)
        of_ref[t] = h2[:, 0:H]
        or_ref[T - 1 - t] = h2[:, H:2 * H]
        if emit_flipped:
            off_ref[T - 1 - t] = h2[:, 0:H]
            orf_ref[t] = h2[:, H:2 * H]
        return h2, c2

    h, c = lax.fori_loop(0, T, body, (h_s[...], c_s[...]), unroll=16)
    h_s[...] = h
    c_s[...] = c


def _bilstm_layer(fwd_arrs, rev_arrs, din, args, emit_flipped):
    n_in = len(fwd_arrs)
    w = din // n_in
    in_specs = ([pl.BlockSpec((T, B, w), lambda i: (i, 0, 0))
                 for _ in range(2 * n_in)])
    operands = list(fwd_arrs) + list(rev_arrs)
    wf, wr, bihb, bhhb, wbd = args
    in_specs += [
        pl.BlockSpec((din, W2), lambda i: (0, 0)),
        pl.BlockSpec((din, W2), lambda i: (0, 0)),
    ]
    operands += [wf, wr]
    pc = pl.pallas_call(
        functools.partial(_proj_body, n_in == 2),
        grid=(NB,),
        in_specs=in_specs,
        out_specs=pl.BlockSpec((T, B, W2), lambda i: (i, 0, 0)),
        out_shape=jax.ShapeDtypeStruct((S, B, W2), jnp.float32),
        compiler_params=_ARB,
    )(*operands)
    out_specs = [
        pl.BlockSpec((T, B, H), lambda i: (i, 0, 0)),
        pl.BlockSpec((T, B, H), lambda i: (NB - 1 - i, 0, 0)),
    ]
    out_shape = [jax.ShapeDtypeStruct((S, B, H), jnp.float32)] * 2
    if emit_flipped:
        out_specs += [
            pl.BlockSpec((T, B, H), lambda i: (NB - 1 - i, 0, 0)),
            pl.BlockSpec((T, B, H), lambda i: (i, 0, 0)),
        ]
        out_shape += [jax.ShapeDtypeStruct((S, B, H), jnp.float32)] * 2
    return pl.pallas_call(
        functools.partial(_rec_body, emit_flipped),
        grid=(NB,),
        in_specs=[
            pl.BlockSpec((T, B, W2), lambda i: (i, 0, 0)),
            pl.BlockSpec((1, W2), lambda i: (0, 0)),
            pl.BlockSpec((1, W2), lambda i: (0, 0)),
            pl.BlockSpec((2 * H, W2), lambda i: (0, 0)),
        ],
        out_specs=out_specs,
        out_shape=out_shape,
        scratch_shapes=[
            pltpu.VMEM((B, 2 * H), jnp.float32),
            pltpu.VMEM((B, 2 * H), jnp.float32),
        ],
        compiler_params=_ARB,
    )(pc, bihb, bhhb, wbd)


def _score_body(f_ref, r_ref, w_ref, b_ref, s2_ref):
    xc = jnp.concatenate([f_ref[...], r_ref[...]], axis=-1).reshape(T * B, D)
    s = jnp.dot(xc, w_ref[...])
    s = jax.nn.sigmoid(s + b_ref[0, 0])
    s2_ref[...] = s.reshape(T, B, D)[:, :, 0]


def _rank_body(sbt_ref, stb_ref, idxl_ref, idxg_ref, loss_ref):
    b = pl.program_id(0)
    s_row = sbt_ref[...].reshape(1, S)
    stb = stb_ref[...]
    bmask = lax.broadcasted_iota(jnp.int32, (1, B), 1) == b
    s_col = jnp.sum(jnp.where(bmask, stb, 0.0), axis=1, keepdims=True)
    sp = lax.broadcast_in_dim(s_col, (S, S), (0, 1))
    sl = lax.broadcast_in_dim(s_row, (S, S), (0, 1))
    pidx = lax.broadcasted_iota(jnp.int32, (S, S), 0)
    iidx = lax.broadcasted_iota(jnp.int32, (S, S), 1)
    ahead = (sp > sl) | ((sp == sl) & (pidx < iidx))
    rank = jnp.sum(ahead.astype(jnp.int32), axis=0, keepdims=True)
    oh = (lax.broadcasted_iota(jnp.int32, (K, S), 0) == rank).astype(jnp.int32)
    iol = lax.broadcasted_iota(jnp.int32, (K, S), 1)
    idxc = jnp.sum(oh * iol, axis=1, keepdims=True)
    idxl_ref[...] = idxc.reshape(1, K, 1)
    idxg_ref[...] = (idxc + b * S).reshape(1, K, 1)

    mu = jnp.mean(s_row)
    dv = s_row - mu
    std = jnp.sqrt(jnp.sum(dv * dv) / (S - 1))

    @pl.when(b == 0)
    def _init():
        loss_ref[...] = jnp.zeros_like(loss_ref)

    loss_ref[...] += std * (1.0 / B)


_NCHUNK = 2
_CW = K // _NCHUNK


def _sc_gather(idxl_flat, idxg_flat, x2, pe2):
    mesh = plsc.VectorSubcoreMesh(core_axis_name="c", subcore_axis_name="s")

    @functools.partial(
        pl.kernel, mesh=mesh,
        out_type=[jax.ShapeDtypeStruct((B * K, D), jnp.float32),
                  jax.ShapeDtypeStruct((B * K, D), jnp.float32)],
        scratch_types=[
            pltpu.VMEM((_CW,), jnp.int32),
            pltpu.VMEM((_CW,), jnp.int32),
            pltpu.VMEM((_CW,), jnp.int32),
            pltpu.VMEM((_CW,), jnp.int32),
            pltpu.VMEM((_CW, D), jnp.float32),
            pltpu.VMEM((_CW, D), jnp.float32),
            pltpu.VMEM((_CW, D), jnp.float32),
            pltpu.VMEM((_CW, D), jnp.float32),
            pltpu.SemaphoreType.DMA,
        ],
    )
    def k(idxl_hbm, idxg_hbm, x_hbm, pe_hbm, gx_hbm, gp_hbm,
          idx_a, idx_b, gidx_a, gidx_b, xr_a, xr_b, pr_a, pr_b, sem):
        b = lax.axis_index("s") * 2 + lax.axis_index("c")
        pltpu.sync_copy(idxl_hbm.at[pl.ds(b * K, _CW)], idx_a)
        pltpu.sync_copy(idxl_hbm.at[pl.ds(b * K + _CW, _CW)], idx_b)
        pltpu.sync_copy(idxg_hbm.at[pl.ds(b * K, _CW)], gidx_a)
        pltpu.sync_copy(idxg_hbm.at[pl.ds(b * K + _CW, _CW)], gidx_b)
        copies = [
            pltpu.async_copy(x_hbm.at[gidx_a], xr_a, sem),
            pltpu.async_copy(x_hbm.at[gidx_b], xr_b, sem),
            pltpu.async_copy(pe_hbm.at[idx_a], pr_a, sem),
            pltpu.async_copy(pe_hbm.at[idx_b], pr_b, sem),
        ]
        for cp in copies:
            cp.wait()
        pltpu.sync_copy(xr_a, gx_hbm.at[pl.ds(b * K, _CW)])
        pltpu.sync_copy(xr_b, gx_hbm.at[pl.ds(b * K + _CW, _CW)])
        pltpu.sync_copy(pr_a, gp_hbm.at[pl.ds(b * K, _CW)])
        pltpu.sync_copy(pr_b, gp_hbm.at[pl.ds(b * K + _CW, _CW)])

    return k(idxl_flat, idxg_flat, x2, pe2)


def kernel(x, pos_emb, W_ih_l0, W_hh_l0, b_ih_l0, b_hh_l0,
           W_ih_l0r, W_hh_l0r, b_ih_l0r, b_hh_l0r,
           W_ih_l1, W_hh_l1, b_ih_l1, b_hh_l1,
           W_ih_l1r, W_hh_l1r, b_ih_l1r, b_hh_l1r,
           lin_w, lin_b):
    f32 = jnp.float32
    xt = jnp.swapaxes(x, 0, 1)

    def stretch(w_t, off):
        din = w_t.shape[0]
        out = jnp.zeros((din, W2), f32)
        for k in range(4):
            out = out.at[:, 128 * k + off:128 * k + off + H].set(
                w_t[:, H * k:H * (k + 1)])
        return out

    def stretch_b(b_f, b_r):
        out = jnp.zeros((1, W2), f32)
        for k in range(4):
            out = out.at[0, 128 * k:128 * k + H].set(b_f[H * k:H * (k + 1)])
            out = out.at[0, 128 * k + H:128 * (k + 1)].set(b_r[H * k:H * (k + 1)])
        return out

    def blockdiag(whh_f_t, whh_r_t):
        out = jnp.zeros((2 * H, W2), f32)
        out = out.at[0:H, :].set(stretch(whh_f_t, 0)[:, :])
        out = out.at[H:2 * H, :].set(stretch(whh_r_t, H)[:, :])
        return out

    def prep(W_ih_f, W_hh_f, b_ih_f, b_hh_f, W_ih_r, W_hh_r, b_ih_r, b_hh_r):
        return (stretch(W_ih_f.T.astype(f32), 0),
                stretch(W_ih_r.T.astype(f32), H),
                stretch_b(b_ih_f, b_ih_r),
                stretch_b(b_hh_f, b_hh_r),
                blockdiag(W_hh_f.T.astype(f32), W_hh_r.T.astype(f32)))

    args0 = prep(W_ih_l0, W_hh_l0, b_ih_l0, b_hh_l0,
                 W_ih_l0r, W_hh_l0r, b_ih_l0r, b_hh_l0r)
    args1 = prep(W_ih_l1, W_hh_l1, b_ih_l1, b_hh_l1,
                 W_ih_l1r, W_hh_l1r, b_ih_l1r, b_hh_l1r)

    xtf = jnp.flip(xt, 0)
    of0, or0, off0, orf0 = _bilstm_layer(
        [xt], [xtf], D, args0, emit_flipped=True)
    of1, or1 = _bilstm_layer(
        [of0, or0], [off0, orf0], D, args1, emit_flipped=False)

    w_pad = jnp.pad(lin_w.T, ((0, 0), (0, D - 1)))
    lb = lin_b.reshape(1, 1)
    s3 = pl.pallas_call(
        _score_body,
        grid=(NB,),
        in_specs=[
            pl.BlockSpec((T, B, H), lambda i: (i, 0, 0)),
            pl.BlockSpec((T, B, H), lambda i: (i, 0, 0)),
            pl.BlockSpec((D, D), lambda i: (0, 0)),
            pl.BlockSpec((1, 1), lambda i: (0, 0)),
        ],
        out_specs=pl.BlockSpec((T, B), lambda i: (i, 0)),
        out_shape=jax.ShapeDtypeStruct((S, B), jnp.float32),
        compiler_params=_ARB,
    )(of1, or1, w_pad, lb)

    stb = s3
    sbt = jnp.swapaxes(stb, 0, 1)
    sbt3 = sbt[:, None, :]

    idxl, idxg, loss = pl.pallas_call(
        _rank_body,
        grid=(B,),
        in_specs=[
            pl.BlockSpec((1, 1, S), lambda b: (b, 0, 0)),
            pl.BlockSpec((S, B), lambda b: (0, 0)),
        ],
        out_specs=[
            pl.BlockSpec((1, K, 1), lambda b: (b, 0, 0)),
            pl.BlockSpec((1, K, 1), lambda b: (b, 0, 0)),
            pl.BlockSpec((1, 1), lambda b: (0, 0)),
        ],
        out_shape=[
            jax.ShapeDtypeStruct((B, K, 1), jnp.int32),
            jax.ShapeDtypeStruct((B, K, 1), jnp.int32),
            jax.ShapeDtypeStruct((1, 1), jnp.float32),
        ],
        compiler_params=_ARB,
    )(sbt3, stb)

    gx, gp = _sc_gather(idxl.reshape(B * K), idxg.reshape(B * K),
                        x.reshape(B * S, D),
                        pos_emb.reshape(S, D))
    gx3 = gx.reshape(B, K, D)
    gp3 = gp.reshape(B, K, D)
    feat = jnp.stack([gx3, gp3], axis=1)

    score = sbt[:, :, None]
    return feat, gp3, loss[0, 0], score
